# baseline (speedup 1.0000x reference)
import jax
import jax.numpy as jnp
from jax.experimental import pallas as pl
from jax.experimental.pallas import tpu as pltpu

_B = 64
_PLANES = 64 * 5 * 26
_NP = 320
_GRID = _PLANES // _NP


def _tc_body(x_ref, o_ref, acc_ref):
    i = pl.program_id(0)

    @pl.when(i == 0)
    def _():
        acc_ref[0, 0] = 0.0

    conf = x_ref[:, :, 4]
    s = 1.0 / (1.0 + jnp.exp(-conf))
    acc_ref[0, 0] += jnp.sum(s * s)

    @pl.when(i == _GRID - 1)
    def _():
        o_ref[0, 0] = acc_ref[0, 0]


_tc_call = pl.pallas_call(
    _tc_body,
    grid=(_GRID,),
    in_specs=[
        pl.BlockSpec((_NP, 26, 85), lambda i: (i, 0, 0), memory_space=pltpu.VMEM)
    ],
    out_specs=pl.BlockSpec((1, 1), lambda i: (0, 0), memory_space=pltpu.SMEM),
    out_shape=jax.ShapeDtypeStruct((1, 1), jnp.float32),
    scratch_shapes=[pltpu.SMEM((1, 1), jnp.float32)],
)


def kernel(predictions, targets):
    pred3 = predictions.reshape(_PLANES, 26, 85)
    out = _tc_call(pred3)
    return out[0, 0] * (1.0 / _B)


# speedup vs baseline: 1.0321x; 1.0321x over previous
import jax
import jax.numpy as jnp
from jax.experimental import pallas as pl
from jax.experimental.pallas import tpu as pltpu

_B = 64
_GRID = 64


def _tc_body(x_ref, o_ref, acc_ref):
    i = pl.program_id(0)

    @pl.when(i == 0)
    def _():
        acc_ref[0, 0] = 0.0

    x = x_ref[...]
    conf = x[0, :, :, :, 4]
    s = 1.0 / (1.0 + jnp.exp(-conf))
    acc_ref[0, 0] += jnp.sum(s * s)

    @pl.when(i == _GRID - 1)
    def _():
        o_ref[0, 0] = acc_ref[0, 0]


_tc_call = pl.pallas_call(
    _tc_body,
    grid=(_GRID,),
    in_specs=[
        pl.BlockSpec(
            (1, 5, 26, 26, 85), lambda i: (i, 0, 0, 0, 0), memory_space=pltpu.VMEM
        )
    ],
    out_specs=pl.BlockSpec((1, 1), lambda i: (0, 0), memory_space=pltpu.SMEM),
    out_shape=jax.ShapeDtypeStruct((1, 1), jnp.float32),
    scratch_shapes=[pltpu.SMEM((1, 1), jnp.float32)],
)


def kernel(predictions, targets):
    out = _tc_call(predictions)
    return out[0, 0] * (1.0 / _B)
